# baseline (speedup 1.0000x reference)
import functools

import jax
import jax.numpy as jnp
from jax.experimental import pallas as pl
from jax.experimental.pallas import tpu as pltpu

B = 1024
N_CAND = 50000
N_FEAT = 96
D_MAIN = 128
D_BLOCK = 256
CTX = 96

NC_PAD = 50176


def _encode_block(x_ref, wi_ref, bi_ref, wk_ref, bk_ref, ck_ref, cksq_ref):
    cx = jnp.dot(x_ref[:], wi_ref[:], preferred_element_type=jnp.float32) + bi_ref[:]
    ck = jnp.dot(cx, wk_ref[:], preferred_element_type=jnp.float32) + bk_ref[:]
    ck_ref[:] = ck
    cksq_ref[:] = jnp.sum(ck * ck, axis=-1, keepdims=True)


def _encode_candidates(cand_p, W_in, b_in, W_K, b_K):
    BLK = 512
    grid = (NC_PAD // BLK,)
    ck, cksq = pl.pallas_call(
        _encode_block,
        grid=grid,
        in_specs=[
            pl.BlockSpec((BLK, N_FEAT), lambda i: (i, 0)),
            pl.BlockSpec((N_FEAT, D_MAIN), lambda i: (0, 0)),
            pl.BlockSpec((1, D_MAIN), lambda i: (0, 0)),
            pl.BlockSpec((D_MAIN, D_MAIN), lambda i: (0, 0)),
            pl.BlockSpec((1, D_MAIN), lambda i: (0, 0)),
        ],
        out_specs=[
            pl.BlockSpec((BLK, D_MAIN), lambda i: (i, 0)),
            pl.BlockSpec((BLK, 1), lambda i: (i, 0)),
        ],
        out_shape=[
            jax.ShapeDtypeStruct((NC_PAD, D_MAIN), jnp.float32),
            jax.ShapeDtypeStruct((NC_PAD, 1), jnp.float32),
        ],
    )(cand_p, W_in, b_in[None, :], W_K, b_K[None, :])
    return ck, cksq


def kernel(x_num, candidate_x_num, candidate_y, W_in, b_in, W_K, b_K, W_le, b_le,
           W_T1, b_T1, W_T2, ln1_g, ln1_b, W_b1, b_b1, W_b2, b_b2, h_g, h_b, W_h, b_h,
           context_size):
    cand_p = jnp.pad(candidate_x_num, ((0, NC_PAD - N_CAND), (0, 0)))
    ck_p, cksq_p = _encode_candidates(cand_p, W_in, b_in, W_K, b_K)
    ck = ck_p[:N_CAND]
    ck_sq = cksq_p[:N_CAND, 0]

    x = x_num @ W_in + b_in
    k = x @ W_K + b_K
    k_sq = jnp.sum(k * k, axis=-1, keepdims=True)
    d2 = k_sq - 2.0 * (k @ ck.T) + ck_sq[None, :]
    _, context_idx = jax.lax.top_k(-d2, CTX)
    context_k = ck[context_idx]
    similarities = (-k_sq
                    + 2.0 * jnp.einsum('bd,bcd->bc', k, context_k)
                    - jnp.sum(context_k * context_k, axis=-1))
    probs = jax.nn.softmax(similarities, axis=-1)
    context_y_emb = candidate_y[context_idx][..., None] @ W_le + b_le
    t_in = k[:, None, :] - context_k
    values = context_y_emb + (jax.nn.relu(t_in @ W_T1 + b_T1) @ W_T2)
    context_x = jnp.einsum('bc,bcd->bd', probs, values)
    x = x + context_x

    def _ln(v, g, bb, eps=1e-5):
        mu = v.mean(-1, keepdims=True)
        var = ((v - mu) ** 2).mean(-1, keepdims=True)
        return (v - mu) / jnp.sqrt(var + eps) * g + bb

    h = _ln(x, ln1_g, ln1_b)
    x = x + (jax.nn.relu(h @ W_b1 + b_b1) @ W_b2 + b_b2)
    out = jax.nn.relu(_ln(x, h_g, h_b)) @ W_h + b_h
    return out
